# baseline (speedup 1.0000x reference)
import functools

import numpy as np
import jax
import jax.numpy as jnp
from jax import lax
from jax.experimental import pallas as pl
from jax.experimental.pallas import tpu as pltpu
from jax.experimental.pallas import tpu_sc as plsc

H = W = 512
B = 8
N = H * W
NT = 32
P = N // NT
ROWS = 32
CH = 128
NCH = P // CH


@functools.lru_cache(maxsize=1)
def _metadata():
    x, y = np.meshgrid(np.arange(W, dtype=np.float32),
                       np.arange(H, dtype=np.float32))
    x = x - np.float32(W / 2.0)
    y = y - np.float32(H / 2.0)
    r = np.sqrt(x * x + y * y) / np.float32(np.sqrt(float(H * H + W * W)))
    th = np.arctan2(y, x)
    r = r * np.sign(np.cos(th)) + np.float32(0.5)
    r = r * np.float32(W - 1)
    t2 = np.arctan(np.tan(th)) + np.float32(np.pi / 2)
    t2 = t2 * np.float32((H - 1) / np.pi)
    qy = t2.reshape(-1).astype(np.float32)
    qx = r.reshape(-1).astype(np.float32)
    fy = np.clip(np.floor(qy), 0, H - 2).astype(np.float32)
    fx = np.clip(np.floor(qx), 0, W - 2).astype(np.float32)
    iy = fy.astype(np.int32)
    ix = fx.astype(np.int32)
    ay = np.clip(qy - fy, 0.0, 1.0).astype(np.float32)
    ax = np.clip(qx - fx, 0.0, 1.0).astype(np.float32)

    perm = np.argsort(iy, kind="stable").astype(np.int32)
    iys, ixs = iy[perm], ix[perm]
    loc = np.zeros(N, np.int32)
    rstart = np.zeros((NT, 16), np.int32)
    for t in range(NT):
        s, e = t * P, (t + 1) * P
        rs = min(int(iys[s]), H - ROWS)
        rstart[t, :] = rs
        assert int(iys[e - 1]) + 1 - rs < ROWS
        loc[s:e] = (iys[s:e] - rs) * W + ixs[s:e]
    return (rstart,
            loc.reshape(NT, P),
            ax[perm].reshape(NT, P).astype(np.float32),
            ay[perm].reshape(NT, P).astype(np.float32),
            perm.reshape(NT, NCH, CH))


def _build_call():
    mesh = plsc.VectorSubcoreMesh(core_axis_name="c", subcore_axis_name="s")

    @functools.partial(
        pl.kernel,
        mesh=mesh,
        compiler_params=pltpu.CompilerParams(needs_layout_passes=False, use_tc_tiling_on_sc=False),
        out_type=jax.ShapeDtypeStruct((B, N, 1), jnp.float32),
        scratch_types=[
            pltpu.VMEM((16,), jnp.int32),
            pltpu.VMEM((ROWS * W,), jnp.float32),
            pltpu.VMEM((P,), jnp.int32),
            pltpu.VMEM((P,), jnp.float32),
            pltpu.VMEM((P,), jnp.float32),
            pltpu.VMEM((NCH, CH), jnp.int32),
            pltpu.VMEM((P, 1), jnp.float32),
            pltpu.SemaphoreType.DMA,
        ],
    )
    def polar2cart(in_hbm, rstart_hbm, loc_hbm, wx_hbm, wy_hbm, perm_hbm,
                   out_hbm, rs_v, img_v, loc_v, wx_v, wy_v, perm_v,
                   acc_v, sem):
        wid = lax.axis_index("s") * 2 + lax.axis_index("c")
        pltpu.sync_copy(rstart_hbm.at[wid], rs_v)
        pltpu.sync_copy(loc_hbm.at[wid], loc_v)
        pltpu.sync_copy(wx_hbm.at[wid], wx_v)
        pltpu.sync_copy(wy_hbm.at[wid], wy_v)
        pltpu.sync_copy(perm_hbm.at[wid], perm_v)

        base0 = rs_v[pl.ds(0, 16)][0] * W

        def batch_body(b, carry):
            pltpu.async_copy(in_hbm.at[pl.ds(b * N + base0, ROWS * W)],
                             img_v, sem).wait()

            def px_body(i, c):
                sl = pl.ds(i * 16, 16)
                l = loc_v[sl]
                v00 = plsc.load_gather(img_v, [l])
                v01 = plsc.load_gather(img_v, [l + 1])
                v10 = plsc.load_gather(img_v, [l + W])
                v11 = plsc.load_gather(img_v, [l + (W + 1)])
                ax = wx_v[sl]
                ay = wy_v[sl]
                top = v00 + ax * (v01 - v00)
                bot = v10 + ax * (v11 - v10)
                res = top + ay * (bot - top)
                pos = i * 16 + lax.iota(jnp.int32, 16)
                plsc.store_scatter(acc_v, [pos, pos * 0], res)
                return c

            lax.fori_loop(0, P // 16, px_body, 0)

            def scat_body(j, c):
                pltpu.async_copy(acc_v.at[pl.ds(j * CH, CH), :],
                                 out_hbm.at[b].at[perm_v.at[j]], sem).wait()
                return c

            lax.fori_loop(0, NCH, scat_body, 0)
            return carry

        lax.fori_loop(0, B, batch_body, 0)

    return polar2cart


def kernel(inputs):
    rstart, loc, wx, wy, perm = _metadata()
    call = _build_call()
    out = call(inputs.reshape(B * N),
               jnp.asarray(rstart), jnp.asarray(loc),
               jnp.asarray(wx), jnp.asarray(wy), jnp.asarray(perm))
    return out.reshape(B, H, W, 1)
